# baseline (speedup 1.0000x reference)
import functools

import jax
import jax.numpy as jnp
from jax import lax
from jax.experimental import pallas as pl
from jax.experimental.pallas import tpu as pltpu
from jax.experimental.pallas import tpu_sc as plsc

_BASE = 10000
_NC = 2
_NS = 16
_NW = _NC * _NS
_L = 16
_CH = 128


def _lane_swap(x, idx2d):
    return lax.gather(
        x, idx2d,
        lax.GatherDimensionNumbers(
            offset_dims=(), collapsed_slice_dims=(0,), start_index_map=(0,)),
        slice_sizes=(1,),
        mode=lax.GatherScatterMode.PROMISE_IN_BOUNDS)


def _rope_coeffs(seq_len: int, dim: int):
    freqs = 1.0 / (_BASE ** (jnp.arange(0, dim, 2, dtype=jnp.float32) / dim))
    ang = jnp.outer(jnp.arange(seq_len, dtype=jnp.float32), freqs)
    cos = jnp.cos(ang)
    sin = jnp.sin(ang)
    a = jnp.stack([cos, cos], axis=-1).reshape(seq_len, dim)
    b = jnp.stack([-sin, sin], axis=-1).reshape(seq_len, dim)
    return a, b


def _make_sc_kernel(rows: int, dim: int, seq_len: int):
    assert rows % (_NW * _CH) == 0
    rpw = rows // _NW
    nchunk = rpw // _CH
    mesh = plsc.VectorSubcoreMesh(core_axis_name="c", subcore_axis_name="s")

    @functools.partial(
        pl.kernel,
        mesh=mesh,
        out_type=jax.ShapeDtypeStruct((rows, dim), jnp.float32),
        compiler_params=pltpu.CompilerParams(use_tc_tiling_on_sc=False),
        scratch_types=[
            pltpu.VMEM((seq_len, dim), jnp.float32),
            pltpu.VMEM((seq_len, dim), jnp.float32),
            pltpu.VMEM((_CH,), jnp.int32),
            pltpu.VMEM((_CH, dim), jnp.float32),
            pltpu.VMEM((_CH, dim), jnp.float32),
            pltpu.SemaphoreType.DMA,
        ],
    )
    def emb(table_hbm, ids_hbm, a_hbm, b_hbm, out_hbm,
            a_v, b_v, idx_v, rows_v, out_v, sem):
        wid = lax.axis_index("s") * _NC + lax.axis_index("c")
        base = wid * rpw
        pltpu.sync_copy(a_hbm, a_v)
        pltpu.sync_copy(b_hbm, b_v)
        swap2d = (lax.iota(jnp.int32, _L) ^ 1).reshape(_L, 1)

        def chunk_body(c, carry):
            row0 = base + c * _CH
            pltpu.sync_copy(ids_hbm.at[pl.ds(row0, _CH)], idx_v)
            pltpu.async_copy(table_hbm.at[idx_v], rows_v, sem).wait()

            def row_body(r, rcarry):
                s = lax.rem(c * _CH + r, seq_len)
                for j in range(dim // _L):
                    sl = pl.ds(j * _L, _L)
                    x = rows_v[r, sl]
                    xsw = _lane_swap(x, swap2d)
                    out_v[r, sl] = x * a_v[s, sl] + xsw * b_v[s, sl]
                return rcarry

            lax.fori_loop(0, _CH, row_body, 0, unroll=False)
            pltpu.sync_copy(out_v, out_hbm.at[pl.ds(row0, _CH)])
            return carry

        lax.fori_loop(0, nchunk, chunk_body, 0, unroll=False)

    return emb


def kernel(token_ids, table):
    bsz, seq_len = token_ids.shape
    vocab, dim = table.shape
    rows = bsz * seq_len
    ids = token_ids.reshape(rows).astype(jnp.int32)
    a, b = _rope_coeffs(seq_len, dim)
    out = _make_sc_kernel(rows, dim, seq_len)(table, ids, a, b)
    return out.reshape(bsz, seq_len, dim)


# speedup vs baseline: 1.1944x; 1.1944x over previous
import functools

import jax
import jax.numpy as jnp
from jax import lax
from jax.experimental import pallas as pl
from jax.experimental.pallas import tpu as pltpu
from jax.experimental.pallas import tpu_sc as plsc

_BASE = 10000
_NC = 2
_NS = 16
_NW = _NC * _NS
_L = 16
_CH = 128


def _lane_swap(x, idx2d):
    return lax.gather(
        x, idx2d,
        lax.GatherDimensionNumbers(
            offset_dims=(), collapsed_slice_dims=(0,), start_index_map=(0,)),
        slice_sizes=(1,),
        mode=lax.GatherScatterMode.PROMISE_IN_BOUNDS)


def _rope_coeffs(seq_len: int, dim: int):
    freqs = 1.0 / (_BASE ** (jnp.arange(0, dim, 2, dtype=jnp.float32) / dim))
    ang = jnp.outer(jnp.arange(seq_len, dtype=jnp.float32), freqs)
    cos = jnp.cos(ang)
    sin = jnp.sin(ang)
    a = jnp.stack([cos, cos], axis=-1).reshape(seq_len, dim)
    b = jnp.stack([-sin, sin], axis=-1).reshape(seq_len, dim)
    return a, b


def _make_sc_kernel(rows: int, dim: int, seq_len: int):
    assert rows % (_NW * _CH) == 0
    rpw = rows // _NW
    nchunk = rpw // _CH
    nchunk_g = rows // _CH
    mesh = plsc.VectorSubcoreMesh(core_axis_name="c", subcore_axis_name="s")

    @functools.partial(
        pl.kernel,
        mesh=mesh,
        out_type=jax.ShapeDtypeStruct((nchunk_g, _CH, dim), jnp.float32),
        compiler_params=pltpu.CompilerParams(use_tc_tiling_on_sc=False),
        scratch_types=[
            pltpu.VMEM((seq_len, dim), jnp.float32),
            pltpu.VMEM((seq_len, dim), jnp.float32),
            pltpu.VMEM((nchunk, _CH), jnp.int32),
            pltpu.VMEM((_CH, dim), jnp.float32),
            pltpu.VMEM((_CH, dim), jnp.float32),
            pltpu.VMEM((_CH, dim), jnp.float32),
            pltpu.VMEM((_CH, dim), jnp.float32),
            pltpu.SemaphoreType.DMA,
            pltpu.SemaphoreType.DMA,
            pltpu.SemaphoreType.DMA,
            pltpu.SemaphoreType.DMA,
        ],
    )
    def emb(table_hbm, ids_hbm, a_hbm, b_hbm, out_hbm,
            a_v, b_v, ids_v, r0, r1, o0, o1, gs0, gs1, ss0, ss1):
        wid = lax.axis_index("s") * _NC + lax.axis_index("c")
        pltpu.sync_copy(ids_hbm.at[wid], ids_v)
        pltpu.sync_copy(a_hbm, a_v)
        pltpu.sync_copy(b_hbm, b_v)
        swap2d = (lax.iota(jnp.int32, _L) ^ 1).reshape(_L, 1)
        cbase = wid * nchunk

        def start_gather(c, rbuf, sem):
            pltpu.async_copy(table_hbm.at[ids_v.at[c]], rbuf, sem)

        def wait_gather(c, rbuf, sem):
            pltpu.make_async_copy(table_hbm.at[ids_v.at[c]], rbuf, sem).wait()

        def start_store(c, obuf, sem):
            pltpu.async_copy(obuf, out_hbm.at[cbase + c], sem)

        def wait_store(c, obuf, sem):
            pltpu.make_async_copy(obuf, out_hbm.at[cbase + c], sem).wait()

        def compute(c, rbuf, obuf):
            def row_body(r, rcarry):
                s = lax.rem(c * _CH + r, seq_len)
                for j in range(dim // _L):
                    sl = pl.ds(j * _L, _L)
                    x = rbuf[r, sl]
                    xsw = _lane_swap(x, swap2d)
                    obuf[r, sl] = x * a_v[s, sl] + xsw * b_v[s, sl]
                return rcarry

            lax.fori_loop(0, _CH, row_body, 0, unroll=2)

        start_gather(0, r0, gs0)
        start_gather(1, r1, gs1)

        def half(cc, k, rbuf, obuf, gsem, ssem):
            wait_gather(k, rbuf, gsem)

            @pl.when(cc > 0)
            def _():
                wait_store(k - 2, obuf, ssem)

            compute(k, rbuf, obuf)
            start_store(k, obuf, ssem)

            @pl.when(k + 2 < nchunk)
            def _():
                start_gather(k + 2, rbuf, gsem)

        def pair_body(cc, carry):
            half(cc, 2 * cc, r0, o0, gs0, ss0)
            half(cc, 2 * cc + 1, r1, o1, gs1, ss1)
            return carry

        lax.fori_loop(0, nchunk // 2, pair_body, 0, unroll=False)
        wait_store(nchunk - 2, o0, ss0)
        wait_store(nchunk - 1, o1, ss1)

    return emb


def kernel(token_ids, table):
    bsz, seq_len = token_ids.shape
    vocab, dim = table.shape
    rows = bsz * seq_len
    rpw = rows // _NW
    ids = token_ids.reshape(_NW, rpw // _CH, _CH).astype(jnp.int32)
    a, b = _rope_coeffs(seq_len, dim)
    out = _make_sc_kernel(rows, dim, seq_len)(table, ids, a, b)
    return out.reshape(bsz, seq_len, dim)


# speedup vs baseline: 1.6110x; 1.3488x over previous
import functools

import jax
import jax.numpy as jnp
from jax import lax
from jax.experimental import pallas as pl
from jax.experimental.pallas import tpu as pltpu
from jax.experimental.pallas import tpu_sc as plsc

_BASE = 10000
_NC = 2
_NS = 16
_NW = _NC * _NS
_L = 16
_CH = 128


def _lane_swap(x, idx2d):
    return lax.gather(
        x, idx2d,
        lax.GatherDimensionNumbers(
            offset_dims=(), collapsed_slice_dims=(0,), start_index_map=(0,)),
        slice_sizes=(1,),
        mode=lax.GatherScatterMode.PROMISE_IN_BOUNDS)


def _rope_coeffs(seq_len: int, dim: int):
    freqs = 1.0 / (_BASE ** (jnp.arange(0, dim, 2, dtype=jnp.float32) / dim))
    ang = jnp.outer(jnp.arange(seq_len, dtype=jnp.float32), freqs)
    cos = jnp.cos(ang)
    sin = jnp.sin(ang)
    a = jnp.stack([cos, cos], axis=-1).reshape(seq_len, dim)
    b = jnp.stack([-sin, sin], axis=-1).reshape(seq_len, dim)
    return a, b


def _make_sc_kernel(rows: int, dim: int, seq_len: int):
    assert rows % (_NW * _CH) == 0
    rpw = rows // _NW
    nchunk = rpw // _CH
    nchunk_g = rows // _CH
    mesh = plsc.VectorSubcoreMesh(core_axis_name="c", subcore_axis_name="s")

    @functools.partial(
        pl.kernel,
        mesh=mesh,
        out_type=jax.ShapeDtypeStruct((nchunk_g, _CH, dim), jnp.float32),
        compiler_params=pltpu.CompilerParams(use_tc_tiling_on_sc=False),
        scratch_types=[
            pltpu.VMEM((2 * seq_len, dim), jnp.float32),
            pltpu.VMEM((2 * seq_len, dim), jnp.float32),
            pltpu.VMEM((nchunk, _CH), jnp.int32),
            pltpu.VMEM((_CH, dim), jnp.float32),
            pltpu.VMEM((_CH, dim), jnp.float32),
            pltpu.VMEM((_CH, dim), jnp.float32),
            pltpu.VMEM((_CH, dim), jnp.float32),
            pltpu.SemaphoreType.DMA,
            pltpu.SemaphoreType.DMA,
            pltpu.SemaphoreType.DMA,
            pltpu.SemaphoreType.DMA,
        ],
    )
    def emb(table_hbm, ids_hbm, a_hbm, b_hbm, out_hbm,
            a_v, b_v, ids_v, r0, r1, o0, o1, gs0, gs1, ss0, ss1):
        wid = lax.axis_index("s") * _NC + lax.axis_index("c")
        pltpu.sync_copy(ids_hbm.at[wid], ids_v)
        pltpu.sync_copy(a_hbm, a_v)
        pltpu.sync_copy(b_hbm, b_v)
        swap2d = (lax.iota(jnp.int32, _L) ^ 1).reshape(_L, 1)
        cbase = wid * nchunk

        def start_gather(c, rbuf, sem):
            pltpu.async_copy(table_hbm.at[ids_v.at[c]], rbuf, sem)

        def wait_gather(c, rbuf, sem):
            pltpu.make_async_copy(table_hbm.at[ids_v.at[c]], rbuf, sem).wait()

        def start_store(c, obuf, sem):
            pltpu.async_copy(obuf, out_hbm.at[cbase + c], sem)

        def wait_store(c, obuf, sem):
            pltpu.make_async_copy(obuf, out_hbm.at[cbase + c], sem).wait()

        def compute(c, rbuf, obuf):
            phase = lax.rem(c * _CH, seq_len)

            @plsc.parallel_loop(0, _CH, 1, unroll=4)
            def row_body(r):
                s = phase + r
                for j in range(dim // _L):
                    sl = pl.ds(j * _L, _L)
                    x = rbuf[r, sl]
                    xsw = _lane_swap(x, swap2d)
                    obuf[r, sl] = x * a_v[s, sl] + xsw * b_v[s, sl]

        start_gather(0, r0, gs0)
        start_gather(1, r1, gs1)

        def half(cc, k, rbuf, obuf, gsem, ssem):
            wait_gather(k, rbuf, gsem)

            @pl.when(cc > 0)
            def _():
                wait_store(k - 2, obuf, ssem)

            compute(k, rbuf, obuf)
            start_store(k, obuf, ssem)

            @pl.when(k + 2 < nchunk)
            def _():
                start_gather(k + 2, rbuf, gsem)

        def pair_body(cc, carry):
            half(cc, 2 * cc, r0, o0, gs0, ss0)
            half(cc, 2 * cc + 1, r1, o1, gs1, ss1)
            return carry

        lax.fori_loop(0, nchunk // 2, pair_body, 0, unroll=False)
        wait_store(nchunk - 2, o0, ss0)
        wait_store(nchunk - 1, o1, ss1)

    return emb


def kernel(token_ids, table):
    bsz, seq_len = token_ids.shape
    vocab, dim = table.shape
    rows = bsz * seq_len
    rpw = rows // _NW
    ids = token_ids.reshape(_NW, rpw // _CH, _CH).astype(jnp.int32)
    a, b = _rope_coeffs(seq_len, dim)
    a = jnp.concatenate([a, a], axis=0)
    b = jnp.concatenate([b, b], axis=0)
    out = _make_sc_kernel(rows, dim, seq_len)(table, ids, a, b)
    return out.reshape(bsz, seq_len, dim)


# speedup vs baseline: 1.6477x; 1.0228x over previous
import functools

import jax
import jax.numpy as jnp
from jax import lax
from jax.experimental import pallas as pl
from jax.experimental.pallas import tpu as pltpu
from jax.experimental.pallas import tpu_sc as plsc

_BASE = 10000
_NC = 2
_NS = 16
_NW = _NC * _NS
_L = 16


def _lane_swap(x, idx2d):
    return lax.gather(
        x, idx2d,
        lax.GatherDimensionNumbers(
            offset_dims=(), collapsed_slice_dims=(0,), start_index_map=(0,)),
        slice_sizes=(1,),
        mode=lax.GatherScatterMode.PROMISE_IN_BOUNDS)


def _rope_coeffs(seq_len: int, dim: int):
    freqs = 1.0 / (_BASE ** (jnp.arange(0, dim, 2, dtype=jnp.float32) / dim))
    ang = jnp.outer(jnp.arange(seq_len, dtype=jnp.float32), freqs)
    cos = jnp.cos(ang)
    sin = jnp.sin(ang)
    a = jnp.stack([cos, cos], axis=-1).reshape(seq_len, dim)
    b = jnp.stack([-sin, sin], axis=-1).reshape(seq_len, dim)
    return a, b


def _make_sc_kernel(bsz: int, seq_len: int, dim: int):
    assert bsz % _NW == 0
    spw = bsz // _NW
    g0 = min(128, (seq_len + 1) // 2 + (-((seq_len + 1) // 2)) % 8)
    g1 = seq_len - g0
    assert 0 < g0 <= 128 and 0 <= g1 <= 128 and g0 % 8 == 0
    mesh = plsc.VectorSubcoreMesh(core_axis_name="c", subcore_axis_name="s")

    @functools.partial(
        pl.kernel,
        mesh=mesh,
        out_type=jax.ShapeDtypeStruct((bsz, seq_len, dim), jnp.float32),
        compiler_params=pltpu.CompilerParams(use_tc_tiling_on_sc=False),
        scratch_types=[
            pltpu.VMEM((seq_len, dim), jnp.float32),
            pltpu.VMEM((seq_len, dim), jnp.float32),
            pltpu.VMEM((spw, seq_len), jnp.int32),
            pltpu.VMEM((seq_len, dim), jnp.float32),
            pltpu.VMEM((seq_len, dim), jnp.float32),
            pltpu.VMEM((seq_len, dim), jnp.float32),
            pltpu.VMEM((seq_len, dim), jnp.float32),
            pltpu.SemaphoreType.DMA,
            pltpu.SemaphoreType.DMA,
            pltpu.SemaphoreType.DMA,
            pltpu.SemaphoreType.DMA,
        ],
    )
    def emb(table_hbm, ids_hbm, a_hbm, b_hbm, out_hbm,
            a_v, b_v, ids_v, r0, r1, o0, o1, gs0, gs1, ss0, ss1):
        wid = lax.axis_index("s") * _NC + lax.axis_index("c")
        pltpu.sync_copy(ids_hbm.at[wid], ids_v)
        pltpu.sync_copy(a_hbm, a_v)
        pltpu.sync_copy(b_hbm, b_v)
        swap2d = (lax.iota(jnp.int32, _L) ^ 1).reshape(_L, 1)
        sbase = wid * spw

        def start_gather(q, rbuf, sem):
            pltpu.async_copy(
                table_hbm.at[ids_v.at[q, pl.ds(0, g0)]],
                rbuf.at[pl.ds(0, g0)], sem)
            if g1:
                pltpu.async_copy(
                    table_hbm.at[ids_v.at[q, pl.ds(g0, g1)]],
                    rbuf.at[pl.ds(g0, g1)], sem)

        def wait_gather(q, rbuf, sem):
            pltpu.make_async_copy(
                table_hbm.at[ids_v.at[q, pl.ds(0, g0)]],
                rbuf.at[pl.ds(0, g0)], sem).wait()
            if g1:
                pltpu.make_async_copy(
                    table_hbm.at[ids_v.at[q, pl.ds(g0, g1)]],
                    rbuf.at[pl.ds(g0, g1)], sem).wait()

        def start_store(q, obuf, sem):
            pltpu.async_copy(obuf, out_hbm.at[sbase + q], sem)

        def wait_store(q, obuf, sem):
            pltpu.make_async_copy(obuf, out_hbm.at[sbase + q], sem).wait()

        def compute(rbuf, obuf):
            @plsc.parallel_loop(0, seq_len, 1, unroll=4)
            def row_body(s):
                for j in range(dim // _L):
                    sl = pl.ds(j * _L, _L)
                    x = rbuf[s, sl]
                    xsw = _lane_swap(x, swap2d)
                    obuf[s, sl] = x * a_v[s, sl] + xsw * b_v[s, sl]

        start_gather(0, r0, gs0)
        start_gather(1, r1, gs1)

        def half(qq, q, rbuf, obuf, gsem, ssem):
            wait_gather(q, rbuf, gsem)

            @pl.when(qq > 0)
            def _():
                wait_store(q - 2, obuf, ssem)

            compute(rbuf, obuf)
            start_store(q, obuf, ssem)

            @pl.when(q + 2 < spw)
            def _():
                start_gather(q + 2, rbuf, gsem)

        def pair_body(qq, carry):
            half(qq, 2 * qq, r0, o0, gs0, ss0)
            half(qq, 2 * qq + 1, r1, o1, gs1, ss1)
            return carry

        lax.fori_loop(0, spw // 2, pair_body, 0, unroll=False)
        wait_store(spw - 2, o0, ss0)
        wait_store(spw - 1, o1, ss1)

    return emb


def kernel(token_ids, table):
    bsz, seq_len = token_ids.shape
    vocab, dim = table.shape
    ids = token_ids.reshape(_NW, bsz // _NW, seq_len).astype(jnp.int32)
    a, b = _rope_coeffs(seq_len, dim)
    return _make_sc_kernel(bsz, seq_len, dim)(table, ids, a, b)
